# Initial kernel scaffold; baseline (speedup 1.0000x reference)
import jax
import jax.numpy as jnp
from jax.experimental import pallas as pl


def kernel(x, tables):
    raise NotImplementedError("write your pallas kernel here")


# speedup vs baseline: 40.1182x; 40.1182x over previous
import numpy as np
import jax
import jax.numpy as jnp
from jax import lax
from jax.experimental import pallas as pl
from jax.experimental.pallas import tpu as pltpu
from jax.experimental.pallas import tpu_sc as plsc
from jax._src import config as _jax_src_config

N_LEVELS = 12
N_FEATURES = 2
HASHMAP_SIZE = 2 ** 19
MASK = np.int32(HASHMAP_SIZE - 1)
BASE_RES = 16
GROWTH = 1.38
RES = [int(np.floor(BASE_RES * GROWTH ** l)) for l in range(N_LEVELS)]
P1 = np.uint32(2654435761).astype(np.int32)
P2 = np.int32(805459861)
N_PTS = 524288
N_OUT = N_LEVELS * N_FEATURES

ROWS_PER_SUPER = 8
N_SUPER = HASHMAP_SIZE // ROWS_PER_SUPER

NW = 32
PTS_PER_W = N_PTS // NW
CHUNK = 512
GROUPS = CHUNK // 16
N_CHUNKS = PTS_PER_W // CHUNK
N_IDX = 8 * CHUNK
FIRE = N_IDX // 128


def _fori32(n, body):
    lax.fori_loop(0, n, lambda i, c: (body(i), c)[1], None, unroll=False)


def _body(x_hbm, tab_hbm, out_hbm, x_v, idx_v, m_v, rows_v, out_v, sem):
    wid = lax.axis_index("s") * np.int32(2) + lax.axis_index("c")
    base = wid * np.int32(PTS_PER_W)
    iota = lax.iota(jnp.int32, 16)
    zero16 = jnp.zeros((16,), jnp.int32)

    def chunk_body(ch):
        cbase = base + ch * np.int32(CHUNK)
        pltpu.sync_copy(x_hbm.at[pl.ds(cbase, CHUNK)], x_v)
        for l in range(N_LEVELS):
            res = np.float32(RES[l])
            lvl_off = np.int32(l * N_SUPER)

            def hash_body(g):
                off = g * np.int32(16)
                rows = off + iota
                xi = plsc.load_gather(x_v, [rows, zero16])
                yi = plsc.load_gather(x_v, [rows, zero16 + np.int32(1)])
                zi = plsc.load_gather(x_v, [rows, zero16 + np.int32(2)])
                fx = (xi * res).astype(jnp.int32)
                fy = (yi * res).astype(jnp.int32)
                fz = (zi * res).astype(jnp.int32)
                hy0 = fy * P1
                hz0 = fz * P2
                hxy = (fx ^ hy0, (fx + np.int32(1)) ^ hy0, fx ^ (hy0 + P1),
                       (fx + np.int32(1)) ^ (hy0 + P1))
                for c in range(8):
                    hz = (hz0 + P2) if (c & 4) else hz0
                    h = (hxy[c & 3] ^ hz) & MASK
                    pos = np.int32(c * CHUNK) + off
                    idx_v[pl.ds(pos, 16)] = (
                        lax.shift_right_logical(h, np.int32(3)) + lvl_off)
                    m_v[pl.ds(pos, 16)] = lax.shift_left(
                        h & np.int32(7), np.int32(1))

            _fori32(GROUPS, hash_body)

            def fire_body(j):
                jo = j * np.int32(128)
                pltpu.async_copy(tab_hbm.at[idx_v.at[pl.ds(jo, 128)]],
                                 rows_v.at[pl.ds(jo, 128)], sem)

            def drain_body(j):
                jo = j * np.int32(128)
                pltpu.make_async_copy(tab_hbm.at[idx_v.at[pl.ds(jo, 128)]],
                                      rows_v.at[pl.ds(jo, 128)], sem).wait()

            _fori32(FIRE, fire_body)
            _fori32(FIRE, drain_body)

            def comb_body(g):
                off = g * np.int32(16)
                rows = off + iota
                xi = plsc.load_gather(x_v, [rows, zero16])
                yi = plsc.load_gather(x_v, [rows, zero16 + np.int32(1)])
                zi = plsc.load_gather(x_v, [rows, zero16 + np.int32(2)])
                xs = xi * res
                ys = yi * res
                zs = zi * res
                wx = xs - xs.astype(jnp.int32).astype(jnp.float32)
                wy = ys - ys.astype(jnp.int32).astype(jnp.float32)
                wz = zs - zs.astype(jnp.int32).astype(jnp.float32)
                one = np.float32(1.0)
                ax = (one - wx, wx)
                ay = (one - wy, wy)
                az = (one - wz, wz)
                wxy = (ax[0] * ay[0], ax[1] * ay[0], ax[0] * ay[1],
                       ax[1] * ay[1])
                acc0 = jnp.zeros((16,), jnp.float32)
                acc1 = jnp.zeros((16,), jnp.float32)
                for c in range(8):
                    wc = wxy[c & 3] * az[(c >> 2) & 1]
                    pos = np.int32(c * CHUNK) + off
                    m0 = m_v[pl.ds(pos, 16)]
                    srow = pos + iota
                    f0 = plsc.load_gather(rows_v, [srow, m0])
                    f1 = plsc.load_gather(rows_v, [srow, m0 + np.int32(1)])
                    acc0 = acc0 + wc * f0
                    acc1 = acc1 + wc * f1
                plsc.store_scatter(out_v, [rows, zero16 + np.int32(2 * l)],
                                   acc0)
                plsc.store_scatter(out_v, [rows, zero16 + np.int32(2 * l + 1)],
                                   acc1)

            _fori32(GROUPS, comb_body)
        pltpu.sync_copy(out_v, out_hbm.at[pl.ds(cbase, CHUNK)])

    _fori32(N_CHUNKS, chunk_body)


@jax.jit
def _hash_grid(x, tables16):
    mesh = plsc.VectorSubcoreMesh(core_axis_name="c", subcore_axis_name="s")
    return pl.kernel(
        _body,
        out_type=jax.ShapeDtypeStruct((N_PTS, N_OUT), jnp.float32),
        mesh=mesh,
        compiler_params=pltpu.CompilerParams(needs_layout_passes=False,
                                             use_tc_tiling_on_sc=False),
        scratch_types=[
            pltpu.VMEM((CHUNK, 3), jnp.float32),
            pltpu.VMEM((N_IDX,), jnp.int32),
            pltpu.VMEM((N_IDX,), jnp.int32),
            pltpu.VMEM((N_IDX, 2 * ROWS_PER_SUPER), jnp.float32),
            pltpu.VMEM((CHUNK, N_OUT), jnp.float32),
            pltpu.SemaphoreType.DMA,
        ],
    )(x, tables16)


def kernel(x, tables):
    x = x.astype(jnp.float32)
    tables16 = tables.astype(jnp.float32).reshape(
        N_LEVELS * N_SUPER, N_FEATURES * ROWS_PER_SUPER)
    with _jax_src_config.enable_x64(False):
        return _hash_grid(x, tables16)
